# baseline (speedup 1.0000x reference)
import functools
import math

import numpy as np
import jax
import jax.numpy as jnp
from jax.experimental import pallas as pl


def _dft_mats(N: int):
    m = np.arange(N)[:, None].astype(np.float64)
    f = np.arange(1, N // 2 + 1)[None, :].astype(np.float64)
    CF = np.cos(2 * np.pi * m * f / N)
    SF = np.sin(2 * np.pi * m * f / N)
    SF[:, -1] = 0.0
    scale = np.where(f[0] == N // 2, 1.0 / N, 2.0 / N)[:, None]
    n = np.arange(N)[None, :].astype(np.float64)
    fc = np.arange(1, N // 2 + 1)[:, None].astype(np.float64)
    iC = scale * np.cos(2 * np.pi * fc * n / N)
    iS = scale * np.sin(2 * np.pi * fc * n / N)
    iS[-1, :] = 0.0
    FW = np.concatenate([CF, SF], axis=1)
    IM = np.concatenate([iC, iS], axis=0)
    return FW.astype(np.float32), IM.astype(np.float32)


def _split_bf16(x):
    h = x.astype(jnp.bfloat16)
    return h, (x - h.astype(jnp.float32)).astype(jnp.bfloat16)


def _dot3(x, mh, ml):
    xh, xl = _split_bf16(x)
    f32 = jnp.float32
    return (jnp.dot(xh, mh, preferred_element_type=f32)
            + jnp.dot(xl, mh, preferred_element_type=f32)
            + jnp.dot(xh, ml, preferred_element_type=f32))


def _body(q_ref, k_ref, v_ref, fw_ref, im_ref, fwh_ref, fwl_ref, imh_ref, iml_ref,
          v_out_ref, c_out_ref, *, topk):
    N = q_ref.shape[-1]
    H = N // 2
    q = q_ref[0, 0]
    k = k_ref[0, 0]
    v = v_ref[0, 0]
    fw = fw_ref[...]
    im = im_ref[...]

    qf = _dot3(q, fwh_ref[...], fwl_ref[...])
    kf = _dot3(k, fwh_ref[...], fwl_ref[...])
    qr, qi = qf[:, :H], qf[:, H:]
    kr, ki = kf[:, :H], kf[:, H:]
    rr = qr * kr + qi * ki
    ri = qi * kr - qr * ki
    dc = (jnp.sum(q, axis=-1, keepdims=True) * jnp.sum(k, axis=-1, keepdims=True)) * (1.0 / N)
    corr = _dot3(jnp.concatenate([rr, ri], axis=-1), imh_ref[...], iml_ref[...]) + dc

    fidx = jax.lax.broadcasted_iota(jnp.int32, corr.shape, 1).astype(jnp.float32)
    work = corr
    ws = []
    for _ in range(topk):
        mx = jnp.max(work, axis=-1, keepdims=True)
        dd = jnp.min(jnp.where(work == mx, fidx, 512.0), axis=-1, keepdims=True)
        ws.append(mx)
        work = jnp.where(fidx == dd, -jnp.inf, work)

    e = jnp.where(work == -jnp.inf, jnp.exp(corr - ws[0]), 0.0)
    denom = jnp.sum(e, axis=-1, keepdims=True)
    oh = e * (1.0 / denom)

    vf = _dot3(v, fwh_ref[...], fwl_ref[...])
    of = _dot3(oh, fwh_ref[...], fwl_ref[...])
    vr, vi = vf[:, :H], vf[:, H:]
    orr, oi = of[:, :H], of[:, H:]
    ar = vr * orr + vi * oi
    ai = vi * orr - vr * oi
    vdc = jnp.sum(v, axis=-1, keepdims=True) * (1.0 / N)
    vagg = _dot3(jnp.concatenate([ar, ai], axis=-1), imh_ref[...], iml_ref[...]) + vdc

    v_out_ref[0, 0] = vagg
    c_out_ref[0] = corr.T


@jax.jit
def kernel(queries, keys, values):
    B, Hh, L, E = queries.shape
    topk = int(math.log(E))
    TL = 4096
    nl = L // TL
    FW, IM = _dft_mats(E)
    fw = jnp.asarray(FW)
    im = jnp.asarray(IM)
    FWh = FW.astype(jnp.bfloat16)
    FWl = (FW - FWh.astype(np.float32)).astype(jnp.bfloat16)
    IMh = IM.astype(jnp.bfloat16)
    IMl = (IM - IMh.astype(np.float32)).astype(jnp.bfloat16)

    grid = (B, Hh, nl)
    mat_spec = pl.BlockSpec((E, E), lambda b, h, lt: (0, 0))
    in_specs = [
        pl.BlockSpec((1, 1, TL, E), lambda b, h, lt: (b, h, lt, 0)),
        pl.BlockSpec((1, 1, TL, E), lambda b, h, lt: (b, h, lt, 0)),
        pl.BlockSpec((1, 1, TL, E), lambda b, h, lt: (b, h, lt, 0)),
        mat_spec, mat_spec, mat_spec, mat_spec, mat_spec, mat_spec,
    ]
    out_specs = [
        pl.BlockSpec((1, 1, TL, E), lambda b, h, lt: (b, h, lt, 0)),
        pl.BlockSpec((1, E, TL), lambda b, h, lt: (b, 0, h * nl + lt)),
    ]
    out_shapes = [
        jax.ShapeDtypeStruct((B, Hh, L, E), jnp.float32),
        jax.ShapeDtypeStruct((B, E, Hh * L), jnp.float32),
    ]
    vagg, corr_m = pl.pallas_call(
        functools.partial(_body, topk=topk),
        grid=grid,
        in_specs=in_specs,
        out_specs=out_specs,
        out_shape=out_shapes,
    )(queries, keys, values, fw, im,
      jnp.asarray(FWh), jnp.asarray(FWl), jnp.asarray(IMh), jnp.asarray(IMl))
    return vagg, corr_m.reshape(B, E, Hh, L)


# speedup vs baseline: 1.2235x; 1.2235x over previous
import functools
import math

import numpy as np
import jax
import jax.numpy as jnp
from jax.experimental import pallas as pl


def _dft_mats(N: int):
    m = np.arange(N)[:, None].astype(np.float64)
    f = np.arange(1, N // 2 + 1)[None, :].astype(np.float64)
    CF = np.cos(2 * np.pi * m * f / N)
    SF = np.sin(2 * np.pi * m * f / N)
    SF[:, -1] = 0.0
    scale = np.where(f[0] == N // 2, 1.0 / N, 2.0 / N)[:, None]
    n = np.arange(N)[None, :].astype(np.float64)
    fc = np.arange(1, N // 2 + 1)[:, None].astype(np.float64)
    iC = scale * np.cos(2 * np.pi * fc * n / N)
    iS = scale * np.sin(2 * np.pi * fc * n / N)
    iS[-1, :] = 0.0
    FW = np.concatenate([CF, SF], axis=1)
    IM = np.concatenate([iC, iS], axis=0)
    return FW.astype(np.float32), IM.astype(np.float32)


def _split_bf16(x):
    h = x.astype(jnp.bfloat16)
    return h, (x - h.astype(jnp.float32)).astype(jnp.bfloat16)


def _dot3(x, mh, ml):
    xh, xl = _split_bf16(x)
    f32 = jnp.float32
    return (jnp.dot(xh, mh, preferred_element_type=f32)
            + jnp.dot(xl, mh, preferred_element_type=f32)
            + jnp.dot(xh, ml, preferred_element_type=f32))


def _body(q_ref, k_ref, v_ref, fw_ref, im_ref, fwh_ref, fwl_ref, imh_ref, iml_ref,
          v_out_ref, c_out_ref, *, topk):
    N = q_ref.shape[-1]
    H = N // 2
    q = q_ref[0, 0]
    k = k_ref[0, 0]
    v = v_ref[0, 0]
    fw = fw_ref[...]
    im = im_ref[...]

    qf = _dot3(q, fwh_ref[...], fwl_ref[...])
    kf = _dot3(k, fwh_ref[...], fwl_ref[...])
    qr, qi = qf[:, :H], qf[:, H:]
    kr, ki = kf[:, :H], kf[:, H:]
    rr = qr * kr + qi * ki
    ri = qi * kr - qr * ki
    dc = (jnp.sum(q, axis=-1, keepdims=True) * jnp.sum(k, axis=-1, keepdims=True)) * (1.0 / N)
    corr = _dot3(jnp.concatenate([rr, ri], axis=-1), imh_ref[...], iml_ref[...]) + dc

    fidx = jax.lax.broadcasted_iota(jnp.int32, corr.shape, 1).astype(jnp.float32)
    work = corr
    ws = []
    for _ in range(topk):
        mx = jnp.max(work, axis=-1, keepdims=True)
        dd = jnp.min(jnp.where(work == mx, fidx, 512.0), axis=-1, keepdims=True)
        ws.append(mx)
        work = jnp.where(fidx == dd, -jnp.inf, work)

    e = jnp.where(work == -jnp.inf, jnp.exp(corr - ws[0]), 0.0)
    denom = jnp.sum(e, axis=-1, keepdims=True)
    oh = e * (1.0 / denom)

    vf = jnp.dot(v, fw, preferred_element_type=jnp.float32)
    of = jnp.dot(oh, fw, preferred_element_type=jnp.float32)
    vr, vi = vf[:, :H], vf[:, H:]
    orr, oi = of[:, :H], of[:, H:]
    ar = vr * orr + vi * oi
    ai = vi * orr - vr * oi
    vdc = jnp.sum(v, axis=-1, keepdims=True) * (1.0 / N)
    vagg = jnp.dot(jnp.concatenate([ar, ai], axis=-1), im,
                   preferred_element_type=jnp.float32) + vdc

    v_out_ref[0, 0] = vagg
    c_out_ref[0] = corr.T


@jax.jit
def kernel(queries, keys, values):
    B, Hh, L, E = queries.shape
    topk = int(math.log(E))
    TL = 4096
    nl = L // TL
    FW, IM = _dft_mats(E)
    fw = jnp.asarray(FW)
    im = jnp.asarray(IM)
    FWh = FW.astype(jnp.bfloat16)
    FWl = (FW - FWh.astype(np.float32)).astype(jnp.bfloat16)
    IMh = IM.astype(jnp.bfloat16)
    IMl = (IM - IMh.astype(np.float32)).astype(jnp.bfloat16)

    grid = (B, Hh, nl)
    mat_spec = pl.BlockSpec((E, E), lambda b, h, lt: (0, 0))
    in_specs = [
        pl.BlockSpec((1, 1, TL, E), lambda b, h, lt: (b, h, lt, 0)),
        pl.BlockSpec((1, 1, TL, E), lambda b, h, lt: (b, h, lt, 0)),
        pl.BlockSpec((1, 1, TL, E), lambda b, h, lt: (b, h, lt, 0)),
        mat_spec, mat_spec, mat_spec, mat_spec, mat_spec, mat_spec,
    ]
    out_specs = [
        pl.BlockSpec((1, 1, TL, E), lambda b, h, lt: (b, h, lt, 0)),
        pl.BlockSpec((1, E, TL), lambda b, h, lt: (b, 0, h * nl + lt)),
    ]
    out_shapes = [
        jax.ShapeDtypeStruct((B, Hh, L, E), jnp.float32),
        jax.ShapeDtypeStruct((B, E, Hh * L), jnp.float32),
    ]
    vagg, corr_m = pl.pallas_call(
        functools.partial(_body, topk=topk),
        grid=grid,
        in_specs=in_specs,
        out_specs=out_specs,
        out_shape=out_shapes,
    )(queries, keys, values, fw, im,
      jnp.asarray(FWh), jnp.asarray(FWl), jnp.asarray(IMh), jnp.asarray(IMl))
    return vagg, corr_m.reshape(B, E, Hh, L)
